# baseline (speedup 1.0000x reference)
import numpy as np
import jax
import jax.numpy as jnp
from jax import lax
from jax.experimental import pallas as pl
from jax.experimental.pallas import tpu as pltpu

_B = 1024
_L1 = 1024
_NUM_FEATURES = 800
_BT = 128
_NBT = _B // _BT

_G = np.zeros((96, 32), dtype=np.float32)
for _dx in range(3):
    for _j in range(10):
        _c = 10 * _j - 1 + _dx
        if 0 <= _c < 96:
            _G[_c, 3 * _j + _dx] = 1.0

_R3 = np.zeros((32, 10, 3), dtype=np.float32)
for _i in range(10):
    for _dy in range(3):
        _R3[3 * _i + _dy, _i, _dy] = 1.0

_L3 = np.zeros((32, 10, 3), dtype=np.float32)
for _j in range(10):
    for _dx in range(3):
        _L3[3 * _j + _dx, _j, _dx] = 1.0

_PERM = np.zeros((_NUM_FEATURES,), dtype=np.int32)
for _i in range(10):
    for _o in range(8):
        for _j in range(10):
            _PERM[_i * 80 + _o * 10 + _j] = _o * 100 + _i * 10 + _j


def _dot_t(x, w):
    return lax.dot_general(x, w, (((1,), (1,)), ((), ())),
                           preferred_element_type=jnp.float32)


def _body(img_hbm, g_ref, w2b_ref, ftwt_ref, ftbt_ref, w1_ref, b1t_ref,
          w2_ref, b2t_ref, w3_ref, b3_ref, out_ref, xbuf, sems):
    k = pl.program_id(0)

    def band_copy(tile, buf, i):
        b0 = tile * _BT
        if i == 0:
            return pltpu.make_async_copy(
                img_hbm.at[pl.ds(b0, _BT), :, pl.ds(0, 2), :],
                xbuf.at[buf, :, :, pl.ds(1, 2), :],
                sems.at[buf, 0])
        return pltpu.make_async_copy(
            img_hbm.at[pl.ds(b0, _BT), :, pl.ds(10 * i - 1, 3), :],
            xbuf.at[buf, :, :, pl.ds(3 * i, 3), :],
            sems.at[buf, i])

    buf = k % 2

    @pl.when(k == 0)
    def _():
        for i in range(10):
            band_copy(0, 0, i).start()

    for i in range(10):
        band_copy(k, buf, i).wait()

    @pl.when(k + 1 < _NBT)
    def _():
        nxt = (k + 1) % 2
        for i in range(10):
            band_copy(k + 1, nxt, i).start()

    xbuf[buf, :, :, pl.ds(0, 1), :] = jnp.zeros((_BT, 3, 1, 96), jnp.float32)
    xbuf[buf, :, :, pl.ds(30, 2), :] = jnp.zeros((_BT, 3, 2, 96), jnp.float32)

    x = xbuf[buf]
    x2d = x.reshape(_BT * 3 * 32, 96)
    y = jnp.dot(x2d, g_ref[...], preferred_element_type=jnp.float32)
    y4 = y.reshape(_BT, 3, 32, 32)
    yt = jnp.transpose(y4, (1, 2, 3, 0)).reshape(3 * 32 * 32, _BT)
    conv_x = jnp.dot(w2b_ref[...], yt, preferred_element_type=jnp.float32)

    bf = jax.nn.sigmoid(10.0 * jnp.clip(conv_x, -1.0, 1.0))
    v = jnp.where(bf > 0.5, bf, 0.0)

    feat = jnp.dot(ftwt_ref[...], v, preferred_element_type=jnp.float32)
    feat = feat + ftbt_ref[...]
    l0 = jnp.clip(feat, 0.0, 1.0)
    s0 = l0[:_L1 // 2, :]
    s1 = l0[_L1 // 2:, :]
    l0c = jnp.concatenate([s0 * s1, s0], axis=0) * (127.0 / 128.0)

    h = jax.nn.relu(
        jnp.dot(w1_ref[...], l0c, preferred_element_type=jnp.float32)
        + b1t_ref[...])
    h = jax.nn.relu(
        jnp.dot(w2_ref[...], h, preferred_element_type=jnp.float32)
        + b2t_ref[...])
    outt = jnp.dot(w3_ref[...], h, preferred_element_type=jnp.float32)
    out_ref[...] = (outt + b3_ref[0, 0]).reshape(1, 8, _BT)


@jax.jit
def kernel(images, conv_w, ft_w, ft_b, w1, b1, w2, b2, w3, b3):
    w2b = jnp.einsum("ocyx,riy,ljx->iojcrl", conv_w, jnp.asarray(_R3),
                     jnp.asarray(_L3)).reshape(800, 3072)
    ftwt = ft_w[jnp.asarray(_PERM)].T

    in_specs = [
        pl.BlockSpec(memory_space=pltpu.MemorySpace.HBM),
        pl.BlockSpec((96, 32), lambda k: (0, 0)),
        pl.BlockSpec((800, 3072), lambda k: (0, 0)),
        pl.BlockSpec((_L1, _NUM_FEATURES), lambda k: (0, 0)),
        pl.BlockSpec((_L1, _BT), lambda k: (0, 0)),
        pl.BlockSpec((15, _L1), lambda k: (0, 0)),
        pl.BlockSpec((15, _BT), lambda k: (0, 0)),
        pl.BlockSpec((32, 15), lambda k: (0, 0)),
        pl.BlockSpec((32, _BT), lambda k: (0, 0)),
        pl.BlockSpec((8, 32), lambda k: (0, 0)),
        pl.BlockSpec(memory_space=pltpu.MemorySpace.SMEM),
    ]
    out = pl.pallas_call(
        _body,
        grid=(_NBT,),
        in_specs=in_specs,
        out_specs=pl.BlockSpec((1, 8, _BT), lambda k: (k, 0, 0)),
        out_shape=jax.ShapeDtypeStruct((_NBT, 8, _BT), jnp.float32),
        scratch_shapes=[
            pltpu.VMEM((2, _BT, 3, 32, 96), jnp.float32),
            pltpu.SemaphoreType.DMA((2, 10)),
        ],
    )(images, jnp.asarray(_G), w2b, ftwt,
      jnp.broadcast_to(ft_b[:, None], (_L1, _BT)), w1,
      jnp.broadcast_to(b1[:, None], (15, _BT)), w2,
      jnp.broadcast_to(b2[:, None], (32, _BT)),
      jnp.pad(w3, ((0, 7), (0, 0))), b3.reshape(1, 1))
    return out[:, 0, :].reshape(_B, 1)


# speedup vs baseline: 1.8872x; 1.8872x over previous
import numpy as np
import jax
import jax.numpy as jnp
from jax import lax
from jax.experimental import pallas as pl
from jax.experimental.pallas import tpu as pltpu

_B = 1024
_L1 = 1024
_NUM_FEATURES = 800
_BT = 128
_NBT = _B // _BT
_LW = 1152

_S = np.zeros((96, 3, 10), dtype=np.float32)
for _dx in range(3):
    for _j in range(10):
        _c = 10 * _j - 1 + _dx
        if 0 <= _c < 96:
            _S[_c, _dx, _j] = 1.0

_PERM = np.zeros((_NUM_FEATURES,), dtype=np.int32)
for _i in range(10):
    for _o in range(8):
        for _j in range(10):
            _PERM[_i * 80 + _o * 10 + _j] = _o * 100 + _i * 10 + _j


def _dot_t(x, w):
    return lax.dot_general(x, w, (((1,), (1,)), ((), ())),
                           preferred_element_type=jnp.float32)


def _body(img_hbm, m_ref, me_ref, m0_ref, ftw_ref, ftb_ref, w1_ref, b1_ref,
          w2_ref, b2_ref, w3_ref, b3_ref, out_ref, xbuf, sems):
    k = pl.program_id(0)

    def band_copy(tile, buf, i, c):
        b0 = tile * _BT
        r0 = 0 if i == 0 else 10 * i - 1
        s = (96 * r0) % 128
        return pltpu.make_async_copy(
            img_hbm.at[pl.ds(b0, _BT), pl.ds(c * 9216 + r0 * 96 - s, 384)],
            xbuf.at[buf, i, :, pl.ds(c * 384, 384)],
            sems.at[buf, i * 3 + c])

    buf = k % 2

    @pl.when(k == 0)
    def _():
        for i in range(10):
            for c in range(3):
                band_copy(0, 0, i, c).start()

    for i in range(10):
        for c in range(3):
            band_copy(k, buf, i, c).wait()

    @pl.when(k + 1 < _NBT)
    def _():
        nxt = (k + 1) % 2
        for i in range(10):
            for c in range(3):
                band_copy(k + 1, nxt, i, c).start()

    m_odd = m_ref[...]
    m_even = me_ref[...]
    m0 = m0_ref[...]
    parts = [
        jnp.dot(xbuf[buf, i],
                m0 if i == 0 else (m_odd if i % 2 == 1 else m_even),
                preferred_element_type=jnp.float32)
        for i in range(10)
    ]
    conv_x = jnp.concatenate(parts, axis=1)

    bf = jax.nn.sigmoid(10.0 * jnp.clip(conv_x, -1.0, 1.0))
    v = jnp.where(bf > 0.5, bf, 0.0)

    feat = jnp.dot(v, ftw_ref[...], preferred_element_type=jnp.float32)
    feat = feat + ftb_ref[...]
    l0 = jnp.clip(feat, 0.0, 1.0)
    s0 = l0[:, :_L1 // 2]
    s1 = l0[:, _L1 // 2:]
    l0c = jnp.concatenate([s0 * s1, s0], axis=1) * (127.0 / 128.0)

    h = jax.nn.relu(_dot_t(l0c, w1_ref[...]) + b1_ref[...])
    h = jax.nn.relu(_dot_t(h, w2_ref[...]) + b2_ref[...])
    out_ref[...] = _dot_t(h, w3_ref[...]) + b3_ref[0, 0]


@jax.jit
def kernel(images, conv_w, ft_w, ft_b, w1, b1, w2, b2, w3, b3):
    m3 = jnp.einsum("ocrx,wxj->crwoj", conv_w, jnp.asarray(_S))
    m3 = m3.reshape(3, 288, 80)
    m = jnp.pad(m3, ((0, 0), (96, 0), (0, 0))).reshape(_LW, 80)
    me = jnp.pad(m3, ((0, 0), (32, 64), (0, 0))).reshape(_LW, 80)
    m03 = jnp.einsum("ocrx,wxj->crwoj", conv_w[:, :, 1:, :], jnp.asarray(_S))
    m03 = m03.reshape(3, 192, 80)
    m0 = jnp.pad(m03, ((0, 0), (0, 192), (0, 0))).reshape(_LW, 80)
    ftw_r = ft_w[jnp.asarray(_PERM)]
    images_flat = images.reshape(_B, 3 * 96 * 96)

    in_specs = [
        pl.BlockSpec(memory_space=pltpu.MemorySpace.HBM),
        pl.BlockSpec((_LW, 80), lambda k: (0, 0)),
        pl.BlockSpec((_LW, 80), lambda k: (0, 0)),
        pl.BlockSpec((_LW, 80), lambda k: (0, 0)),
        pl.BlockSpec((_NUM_FEATURES, _L1), lambda k: (0, 0)),
        pl.BlockSpec((1, _L1), lambda k: (0, 0)),
        pl.BlockSpec((15, _L1), lambda k: (0, 0)),
        pl.BlockSpec((1, 15), lambda k: (0, 0)),
        pl.BlockSpec((32, 15), lambda k: (0, 0)),
        pl.BlockSpec((1, 32), lambda k: (0, 0)),
        pl.BlockSpec((128, 32), lambda k: (0, 0)),
        pl.BlockSpec(memory_space=pltpu.MemorySpace.SMEM),
    ]
    out = pl.pallas_call(
        _body,
        grid=(_NBT,),
        in_specs=in_specs,
        out_specs=pl.BlockSpec((_BT, 128), lambda k: (k, 0)),
        out_shape=jax.ShapeDtypeStruct((_B, 128), jnp.float32),
        scratch_shapes=[
            pltpu.VMEM((2, 10, _BT, _LW), jnp.float32),
            pltpu.SemaphoreType.DMA((2, 30)),
        ],
    )(images_flat, m, me, m0, ftw_r, ft_b.reshape(1, _L1), w1,
      b1.reshape(1, 15),
      w2, b2.reshape(1, 32), jnp.pad(w3, ((0, 127), (0, 0))),
      b3.reshape(1, 1))
    return out[:, :1]
